# baseline (speedup 1.0000x reference)
import functools

import jax
import jax.numpy as jnp
from jax import lax
from jax.experimental import pallas as pl
from jax.experimental.pallas import tpu as pltpu
from jax.experimental.pallas import tpu_sc as plsc

_N = 1000
_B = 256
_ROW = 256 * 256
_TOTAL = _N * _ROW
_BOUND = _B * _ROW
_NW = 32
_W_COMP = 8
_W_MEM = _NW - _W_COMP
_SZ_COMP = _BOUND // _W_COMP
_SZ_MEM = (_TOTAL - _BOUND) // _W_MEM
_CHUNK = 16384
_NC_COMP = _SZ_COMP // _CHUNK
_NC_MEM = _SZ_MEM // _CHUNK
_NBUF = 7
_RAHEAD = 3


def _stream_range(src_hbm, out_hbm, base, nchunks, bufs, rsems, wsems):
    def rd(i, s):
        return pltpu.make_async_copy(
            src_hbm.at[pl.ds(base + i * _CHUNK, _CHUNK)], bufs[s], rsems[s])

    def wr(i, s):
        return pltpu.make_async_copy(
            bufs[s], out_hbm.at[pl.ds(base + i * _CHUNK, _CHUNK)], wsems[s])

    for i in range(min(_RAHEAD, nchunks)):
        rd(i, i % _NBUF).start()
    for i in range(nchunks):
        s = i % _NBUF
        rd(i, s).wait()
        wr(i, s).start()
        ni = i + _RAHEAD
        if ni < nchunks:
            ns = ni % _NBUF
            if ni >= _NBUF:
                wr(ni - _NBUF, ns).wait()
            rd(ni, ns).start()
    for i in range(max(nchunks - _NBUF, 0), nchunks):
        wr(i, i % _NBUF).wait()


def _enqueue_body(comp_hbm, mem_hbm, out_hbm,
                  b0, b1, b2, b3, b4, b5, b6,
                  r0, r1, r2, r3, r4, r5, r6,
                  w0, w1, w2, w3, w4, w5, w6):
    wid = lax.axis_index("s") * 2 + lax.axis_index("c")
    bufs = (b0, b1, b2, b3, b4, b5, b6)
    rsems = (r0, r1, r2, r3, r4, r5, r6)
    wsems = (w0, w1, w2, w3, w4, w5, w6)

    @pl.when(wid < _W_COMP)
    def _():
        _stream_range(comp_hbm, out_hbm, wid * _SZ_COMP, _NC_COMP,
                      bufs, rsems, wsems)

    @pl.when(wid >= _W_COMP)
    def _():
        base = _BOUND + (wid - _W_COMP) * _SZ_MEM
        _stream_range(mem_hbm, out_hbm, base, _NC_MEM, bufs, rsems, wsems)


def kernel(memory_bank, components):
    comps = jax.lax.stop_gradient(components)
    mesh = plsc.VectorSubcoreMesh(core_axis_name="c", subcore_axis_name="s")
    run = functools.partial(
        pl.kernel,
        out_type=jax.ShapeDtypeStruct((_TOTAL,), jnp.float32),
        mesh=mesh,
        scratch_types=(
            [pltpu.VMEM((_CHUNK,), jnp.float32)] * _NBUF
            + [pltpu.SemaphoreType.DMA] * (2 * _NBUF)
        ),
    )(_enqueue_body)
    flat = run(comps.reshape(_BOUND), memory_bank.reshape(_TOTAL))
    return flat.reshape(_N, 256, 256)


# speedup vs baseline: 3.7600x; 3.7600x over previous
import jax
import jax.numpy as jnp
from jax.experimental import pallas as pl
from jax.experimental.pallas import tpu as pltpu

_N = 1000
_B = 256
_RB = 40
_GRID = _N // _RB
_IS = _B // _RB
_CS = _B - _IS * _RB


def _enqueue_kernel(comp_hbm, mem_hbm, out_ref, sem0, sem1):
    i = pl.program_id(0)

    @pl.when(i < _IS)
    def _():
        pltpu.make_async_copy(
            comp_hbm.at[pl.ds(i * _RB, _RB)], out_ref, sem0).start()
        pltpu.make_async_copy(
            comp_hbm.at[pl.ds(i * _RB, _RB)], out_ref, sem0).wait()

    @pl.when(i == _IS)
    def _():
        c0 = pltpu.make_async_copy(
            comp_hbm.at[pl.ds(_IS * _RB, _CS)], out_ref.at[pl.ds(0, _CS)], sem0)
        c1 = pltpu.make_async_copy(
            mem_hbm.at[pl.ds(_B, _RB - _CS)], out_ref.at[pl.ds(_CS, _RB - _CS)],
            sem1)
        c0.start()
        c1.start()
        c0.wait()
        c1.wait()

    @pl.when(i > _IS)
    def _():
        pltpu.make_async_copy(
            mem_hbm.at[pl.ds(i * _RB, _RB)], out_ref, sem0).start()
        pltpu.make_async_copy(
            mem_hbm.at[pl.ds(i * _RB, _RB)], out_ref, sem0).wait()


def kernel(memory_bank, components):
    comps = jax.lax.stop_gradient(components)
    return pl.pallas_call(
        _enqueue_kernel,
        grid=(_GRID,),
        in_specs=[
            pl.BlockSpec(memory_space=pltpu.MemorySpace.HBM),
            pl.BlockSpec(memory_space=pltpu.MemorySpace.HBM),
        ],
        out_specs=pl.BlockSpec((_RB, 256, 256), lambda i: (i, 0, 0)),
        out_shape=jax.ShapeDtypeStruct((_N, 256, 256), memory_bank.dtype),
        scratch_shapes=[pltpu.SemaphoreType.DMA, pltpu.SemaphoreType.DMA],
    )(comps, memory_bank)
